# baseline (speedup 1.0000x reference)
import functools

import jax
import jax.numpy as jnp
import numpy as np
from jax import lax
from jax.experimental import pallas as pl
from jax.experimental.pallas import tpu as pltpu
from jax.experimental.pallas import tpu_sc as plsc

_B, _L, _D = 1024, 200, 64
_NW = 32
_NTC = _B // 128
_NU = _L * _NTC
_UPW = _NU // _NW


def _pe01_table():
    position = jnp.arange(0, _L, dtype=jnp.float32)[:, None]
    div_term = jnp.exp(
        jnp.arange(0, _D, 2, dtype=jnp.float32) * -(np.log(10000.0) / _D))
    pe = jnp.zeros((_L, _D), dtype=jnp.float32)
    pe = pe.at[:, 0::2].set(jnp.sin(position * div_term))
    pe = pe.at[:, 1::2].set(jnp.cos(position * div_term))
    return 0.01 * pe


_mesh = plsc.VectorSubcoreMesh(core_axis_name="c", subcore_axis_name="s")


@functools.partial(
    pl.kernel,
    mesh=_mesh,
    compiler_params=pltpu.CompilerParams(use_tc_tiling_on_sc=False,
                                         needs_layout_passes=False),
    out_type=[jax.ShapeDtypeStruct((_L, 8, _NTC, 8, 128), jnp.float32)] * 3,
    scratch_types=[
        pltpu.VMEM((8, _B), jnp.int32),
        pltpu.VMEM((8, _D, 16), jnp.float32),
        pltpu.VMEM((8, _D, 16), jnp.float32),
        pltpu.VMEM((128, _D), jnp.float32),
        pltpu.VMEM((128, _D), jnp.float32),
        pltpu.VMEM((8, 8, 128), jnp.float32),
        pltpu.VMEM((8, 8, 128), jnp.float32),
        pltpu.VMEM((8, 8, 128), jnp.float32),
        pltpu.VMEM((8, 8, 128), jnp.float32),
        pltpu.VMEM((8, 8, 128), jnp.float32),
        pltpu.VMEM((8, 8, 128), jnp.float32),
        pltpu.SemaphoreType.DMA,
        pltpu.SemaphoreType.DMA,
        pltpu.SemaphoreType.DMA,
        pltpu.SemaphoreType.DMA,
        pltpu.SemaphoreType.DMA,
        pltpu.SemaphoreType.DMA,
        pltpu.SemaphoreType.DMA,
        pltpu.SemaphoreType.DMA,
    ],
)
def _emb_kernel(idxT_hbm, table_hbm, poss_hbm, pes_hbm,
                x_hbm, x1_hbm, x2_hbm,
                idxT_v, pos_v, pe_v, rows0, rows1,
                xs0, xs1, x1s0, x1s1, x2s0, x2s1,
                g0, g1, wx0, wx1, w10, w11, w20, w21):
    wid = lax.axis_index("s") * 2 + lax.axis_index("c")
    ubase = wid * _UPW
    l_stage = jnp.minimum(lax.shift_right_logical(ubase, 3), _L - 8)
    pltpu.sync_copy(idxT_hbm.at[pl.ds(l_stage, 8)], idxT_v)
    pltpu.sync_copy(poss_hbm.at[pl.ds(l_stage, 8)], pos_v)
    pltpu.sync_copy(pes_hbm.at[pl.ds(l_stage, 8)], pe_v)

    rows = (rows0, rows1)
    xs = (xs0, xs1)
    x1s = (x1s0, x1s1)
    x2s = (x2s0, x2s1)
    gsem = (g0, g1)
    xsem = (wx0, wx1)
    x1sem = (w10, w11)
    x2sem = (w20, w21)

    def unit_coords(u):
        ug = ubase + u
        l = lax.shift_right_logical(ug, 3)
        tc = jnp.bitwise_and(ug, 7)
        return l, tc

    def gather_copy(u, p):
        l, tc = unit_coords(u)
        l_loc = l - l_stage
        idx_ref = idxT_v.at[l_loc, pl.ds(tc * 128, 128)]
        return pltpu.make_async_copy(
            table_hbm.at[idx_ref], rows[p], gsem[p])

    def write_copies(u, p):
        l, tc = unit_coords(u)
        return (
            pltpu.make_async_copy(xs[p], x_hbm.at[l, :, tc], xsem[p]),
            pltpu.make_async_copy(x1s[p], x1_hbm.at[l, :, tc], x1sem[p]),
            pltpu.make_async_copy(x2s[p], x2_hbm.at[l, :, tc], x2sem[p]),
        )

    gather_copy(0, 0).start()
    gather_copy(1, 1).start()

    def outer(i, carry):
        for p in range(2):
            u = i * 2 + p
            l, tc = unit_coords(u)
            l_loc = l - l_stage
            gather_copy(u, p).wait()

            @pl.when(u > 1)
            def _wait_prev_writes():
                for c in write_copies(u, p):
                    c.wait()

            for g in range(8):
                rowi = lax.iota(jnp.int32, 16) + 16 * g
                sl = pl.ds(16 * g, 16)

                def col_body(r, rc, g=g, rowi=rowi, sl=sl):
                    for tr in range(8):
                        cf = tr * 8 + r
                        coli = jnp.full((16,), 0, jnp.int32) + cf
                        t = plsc.load_gather(rows[p], [rowi, coli])
                        x1s[p][tr, r, sl] = t
                        xs[p][tr, r, sl] = t + pos_v[l_loc, cf, :]
                        x2s[p][tr, r, sl] = t + pe_v[l_loc, cf, :]
                    return rc

                lax.fori_loop(0, 8, col_body, 0)

            for c in write_copies(u, p):
                c.start()

            @pl.when(u + 2 < _UPW)
            def _prefetch_next():
                gather_copy(u + 2, p).start()

        return carry

    lax.fori_loop(0, _UPW // 2, outer, 0)
    for p in range(2):
        for c in write_copies(_UPW - 2 + p, p):
            c.wait()


def kernel(batch_seqs, token_table, pos_weight):
    idxT = batch_seqs.astype(jnp.int32).T
    pe01 = _pe01_table()
    pos_splat = jnp.broadcast_to(pos_weight[:, :, None], (_L, _D, 16))
    pe_splat = jnp.broadcast_to(pe01[:, :, None], (_L, _D, 16))
    x5, x15, x25 = _emb_kernel(idxT, token_table, pos_splat, pe_splat)

    def fin(o):
        return o.transpose(2, 4, 0, 1, 3).reshape(_B, _L, _D)

    return fin(x5), fin(x15), fin(x25)


# speedup vs baseline: 1.0480x; 1.0480x over previous
import functools

import jax
import jax.numpy as jnp
import numpy as np
from jax import lax
from jax.experimental import pallas as pl
from jax.experimental.pallas import tpu as pltpu
from jax.experimental.pallas import tpu_sc as plsc

_B, _L, _D = 1024, 200, 64
_NW = 32
_NTC = _B // 128
_NU = _L * _NTC
_UPW = _NU // _NW


def _pe01_table():
    position = jnp.arange(0, _L, dtype=jnp.float32)[:, None]
    div_term = jnp.exp(
        jnp.arange(0, _D, 2, dtype=jnp.float32) * -(np.log(10000.0) / _D))
    pe = jnp.zeros((_L, _D), dtype=jnp.float32)
    pe = pe.at[:, 0::2].set(jnp.sin(position * div_term))
    pe = pe.at[:, 1::2].set(jnp.cos(position * div_term))
    return 0.01 * pe


_mesh = plsc.VectorSubcoreMesh(core_axis_name="c", subcore_axis_name="s")


@functools.partial(
    pl.kernel,
    mesh=_mesh,
    compiler_params=pltpu.CompilerParams(use_tc_tiling_on_sc=False,
                                         needs_layout_passes=False),
    out_type=[jax.ShapeDtypeStruct((_L, 8, _NTC, 8, 128), jnp.float32)] * 3,
    scratch_types=[
        pltpu.VMEM((8, _B), jnp.int32),
        pltpu.VMEM((8, _D, 16), jnp.float32),
        pltpu.VMEM((8, _D, 16), jnp.float32),
        pltpu.VMEM((128, _D), jnp.float32),
        pltpu.VMEM((128, _D), jnp.float32),
        pltpu.VMEM((8, 8, 128), jnp.float32),
        pltpu.VMEM((8, 8, 128), jnp.float32),
        pltpu.VMEM((8, 8, 128), jnp.float32),
        pltpu.VMEM((8, 8, 128), jnp.float32),
        pltpu.VMEM((8, 8, 128), jnp.float32),
        pltpu.VMEM((8, 8, 128), jnp.float32),
        pltpu.SemaphoreType.DMA,
        pltpu.SemaphoreType.DMA,
        pltpu.SemaphoreType.DMA,
        pltpu.SemaphoreType.DMA,
        pltpu.SemaphoreType.DMA,
        pltpu.SemaphoreType.DMA,
        pltpu.SemaphoreType.DMA,
        pltpu.SemaphoreType.DMA,
    ],
)
def _emb_kernel(idxT_hbm, table_hbm, poss_hbm, pes_hbm,
                x_hbm, x1_hbm, x2_hbm,
                idxT_v, pos_v, pe_v, rows0, rows1,
                xs0, xs1, x1s0, x1s1, x2s0, x2s1,
                g0, g1, wx0, wx1, w10, w11, w20, w21):
    wid = lax.axis_index("s") * 2 + lax.axis_index("c")
    ubase = wid * _UPW
    l_stage = jnp.minimum(lax.shift_right_logical(ubase, 3), _L - 8)
    pltpu.sync_copy(idxT_hbm.at[pl.ds(l_stage, 8)], idxT_v)
    pltpu.sync_copy(poss_hbm.at[pl.ds(l_stage, 8)], pos_v)
    pltpu.sync_copy(pes_hbm.at[pl.ds(l_stage, 8)], pe_v)

    rows = (rows0, rows1)
    xs = (xs0, xs1)
    x1s = (x1s0, x1s1)
    x2s = (x2s0, x2s1)
    gsem = (g0, g1)
    xsem = (wx0, wx1)
    x1sem = (w10, w11)
    x2sem = (w20, w21)

    def unit_coords(u):
        ug = ubase + u
        l = lax.shift_right_logical(ug, 3)
        tc = jnp.bitwise_and(ug, 7)
        return l, tc

    def gather_copy(u, p):
        l, tc = unit_coords(u)
        l_loc = l - l_stage
        idx_ref = idxT_v.at[l_loc, pl.ds(tc * 128, 128)]
        return pltpu.make_async_copy(
            table_hbm.at[idx_ref], rows[p], gsem[p])

    def write_copies(u, p):
        l, tc = unit_coords(u)
        return (
            pltpu.make_async_copy(xs[p], x_hbm.at[l, :, tc], xsem[p]),
            pltpu.make_async_copy(x1s[p], x1_hbm.at[l, :, tc], x1sem[p]),
            pltpu.make_async_copy(x2s[p], x2_hbm.at[l, :, tc], x2sem[p]),
        )

    gather_copy(0, 0).start()
    gather_copy(1, 1).start()

    def outer(i, carry):
        for p in range(2):
            u = i * 2 + p
            l, tc = unit_coords(u)
            l_loc = l - l_stage
            gather_copy(u, p).wait()

            @pl.when(u > 1)
            def _wait_prev_writes():
                for c in write_copies(u, p):
                    c.wait()

            def grp_body(g, gc):
                rowi = lax.iota(jnp.int32, 16) + 16 * g
                sl = pl.ds(16 * g, 16)
                for cf in range(_D):
                    coli = jnp.full((16,), cf, jnp.int32)
                    t = plsc.load_gather(rows[p], [rowi, coli])
                    x1s[p][cf // 8, cf % 8, sl] = t
                    xs[p][cf // 8, cf % 8, sl] = t + pos_v[l_loc, cf, :]
                    x2s[p][cf // 8, cf % 8, sl] = t + pe_v[l_loc, cf, :]
                return gc

            lax.fori_loop(0, 8, grp_body, 0)

            for c in write_copies(u, p):
                c.start()

            @pl.when(u + 2 < _UPW)
            def _prefetch_next():
                gather_copy(u + 2, p).start()

        return carry

    lax.fori_loop(0, _UPW // 2, outer, 0)
    for p in range(2):
        for c in write_copies(_UPW - 2 + p, p):
            c.wait()


def kernel(batch_seqs, token_table, pos_weight):
    idxT = batch_seqs.astype(jnp.int32).T
    pe01 = _pe01_table()
    pos_splat = jnp.broadcast_to(pos_weight[:, :, None], (_L, _D, 16))
    pe_splat = jnp.broadcast_to(pe01[:, :, None], (_L, _D, 16))
    x5, x15, x25 = _emb_kernel(idxT, token_table, pos_splat, pe_splat)

    def fin(o):
        return o.transpose(2, 4, 0, 1, 3).reshape(_B, _L, _D)

    return fin(x5), fin(x15), fin(x25)


# speedup vs baseline: 1.2064x; 1.1512x over previous
import functools

import jax
import jax.numpy as jnp
import numpy as np
from jax import lax
from jax.experimental import pallas as pl
from jax.experimental.pallas import tpu as pltpu
from jax.experimental.pallas import tpu_sc as plsc

_B, _L, _D = 1024, 200, 64
_NW = 32
_SEQ_PER_W = _B // _NW
_C0, _C1 = 104, 96


def _pe01_table():
    position = jnp.arange(0, _L, dtype=jnp.float32)[:, None]
    div_term = jnp.exp(
        jnp.arange(0, _D, 2, dtype=jnp.float32) * -(np.log(10000.0) / _D))
    pe = jnp.zeros((_L, _D), dtype=jnp.float32)
    pe = pe.at[:, 0::2].set(jnp.sin(position * div_term))
    pe = pe.at[:, 1::2].set(jnp.cos(position * div_term))
    return 0.01 * pe


_mesh = plsc.VectorSubcoreMesh(core_axis_name="c", subcore_axis_name="s")


@functools.partial(
    pl.kernel,
    mesh=_mesh,
    compiler_params=pltpu.CompilerParams(use_tc_tiling_on_sc=False),
    out_type=[jax.ShapeDtypeStruct((_B * _L, _D), jnp.float32)] * 3,
    scratch_types=[
        pltpu.VMEM((_SEQ_PER_W * _L,), jnp.int32),
        pltpu.VMEM((_L, _D), jnp.float32),
        pltpu.VMEM((_L, _D), jnp.float32),
        pltpu.VMEM((_L, _D), jnp.float32),
        pltpu.VMEM((_L, _D), jnp.float32),
        pltpu.VMEM((_L, _D), jnp.float32),
        pltpu.VMEM((_L, _D), jnp.float32),
        pltpu.VMEM((2 * _L, _D), jnp.float32),
        pltpu.SemaphoreType.DMA,
        pltpu.SemaphoreType.DMA,
        pltpu.SemaphoreType.DMA,
        pltpu.SemaphoreType.DMA,
        pltpu.SemaphoreType.DMA,
        pltpu.SemaphoreType.DMA,
        pltpu.SemaphoreType.DMA,
        pltpu.SemaphoreType.DMA,
    ],
)
def _emb_kernel(idx_hbm, table_hbm, pos2_hbm,
                x_hbm, x1_hbm, x2_hbm,
                idx_all, tok0, tok1, xa0, xa1, xb0, xb1, pos2_v,
                g0, g1, s1a, s1b, sxa, sxb, s2a, s2b):
    wid = lax.axis_index("s") * 2 + lax.axis_index("c")
    base_all = wid * (_SEQ_PER_W * _L)
    pltpu.sync_copy(idx_hbm.at[pl.ds(base_all, _SEQ_PER_W * _L)], idx_all)
    pltpu.sync_copy(pos2_hbm, pos2_v)

    toks = (tok0, tok1)
    xs = (xa0, xa1)
    x2s = (xb0, xb1)
    gsem = (g0, g1)
    s1sem = (s1a, s1b)
    xsem = (sxa, sxb)
    x2sem = (s2a, s2b)

    def gather_copies(s, p):
        off = s * _L
        c0 = pltpu.make_async_copy(
            table_hbm.at[idx_all.at[pl.ds(off, _C0)]],
            toks[p].at[pl.ds(0, _C0)], gsem[p])
        c1 = pltpu.make_async_copy(
            table_hbm.at[idx_all.at[pl.ds(off + _C0, _C1)]],
            toks[p].at[pl.ds(_C0, _C1)], gsem[p])
        return c0, c1

    def issue_gather(s, p):
        for c in gather_copies(s, p):
            c.start()

    def wait_gather(s, p):
        for c in gather_copies(s, p):
            c.wait()

    issue_gather(0, 0)
    issue_gather(1, 1)

    def outer(i, carry):
        for p in range(2):
            s = i * 2 + p
            gbase = base_all + s * _L
            wait_gather(s, p)
            cp1 = pltpu.make_async_copy(
                toks[p], x1_hbm.at[pl.ds(gbase, _L)], s1sem[p])
            cp1.start()

            @pl.when(i > 0)
            def _wait_prev_writes():
                pltpu.make_async_copy(
                    xs[p], x_hbm.at[pl.ds(gbase, _L)], xsem[p]).wait()
                pltpu.make_async_copy(
                    x2s[p], x2_hbm.at[pl.ds(gbase, _L)], x2sem[p]).wait()

            def row_body(r, rc):
                for c in range(_D // 16):
                    sl = pl.ds(c * 16, 16)
                    t = toks[p][r, sl]
                    xs[p][r, sl] = t + pos2_v[r, sl]
                    x2s[p][r, sl] = t + pos2_v[_L + r, sl]
                return rc

            lax.fori_loop(0, _L, row_body, 0)
            pltpu.make_async_copy(
                xs[p], x_hbm.at[pl.ds(gbase, _L)], xsem[p]).start()
            pltpu.make_async_copy(
                x2s[p], x2_hbm.at[pl.ds(gbase, _L)], x2sem[p]).start()
            cp1.wait()

            @pl.when(i < (_SEQ_PER_W // 2 - 1))
            def _prefetch_next():
                issue_gather(s + 2, p)

        return carry

    lax.fori_loop(0, _SEQ_PER_W // 2, outer, 0)
    for p in range(2):
        pltpu.make_async_copy(
            xs[p], x_hbm.at[pl.ds(base_all, _L)], xsem[p]).wait()
        pltpu.make_async_copy(
            x2s[p], x2_hbm.at[pl.ds(base_all, _L)], x2sem[p]).wait()


def kernel(batch_seqs, token_table, pos_weight):
    idx = batch_seqs.reshape(-1).astype(jnp.int32)
    pos2 = jnp.concatenate([pos_weight, _pe01_table()], axis=0)
    x, x1, x2 = _emb_kernel(idx, token_table, pos2)
    shape = (_B, _L, _D)
    return x.reshape(shape), x1.reshape(shape), x2.reshape(shape)
